# baseline (device time: 71620 ns/iter reference)
import jax
import jax.numpy as jnp
from jax import lax
from jax.experimental import pallas as pl
from jax.experimental.pallas import tpu as pltpu

N_DEV = 16
T = 16
D = 5


def kernel(x, w_mat):
    m, k_per = x.shape
    k, n = w_mat.shape
    m_per = m // N_DEV
    nt = n // T

    def body(x_ref, w_ref, o_ref, xg_ref, wring, dma_sems,
             send_sems, recv_sems):
        t = pl.program_id(0)
        me = lax.axis_index("i")

        def w_dma(c, slot):
            return pltpu.make_async_copy(
                w_ref.at[:, pl.ds(c * nt, nt)],
                wring.at[slot],
                dma_sems.at[slot],
            )

        @pl.when(t == 0)
        def _():
            for c in range(D):
                w_dma(c, c).start()
            barrier_sem = pltpu.get_barrier_semaphore()
            for d in range(1, N_DEV):
                pl.semaphore_signal(
                    barrier_sem, inc=1,
                    device_id=(lax.rem(me + d, N_DEV),),
                    device_id_type=pl.DeviceIdType.MESH,
                )
            pl.semaphore_wait(barrier_sem, N_DEV - 1)
            for d in range(1, N_DEV):
                j = lax.rem(me + d, N_DEV)
                pltpu.make_async_remote_copy(
                    src_ref=x_ref.at[pl.ds(j * m_per, m_per), :],
                    dst_ref=xg_ref.at[:, pl.ds(me * k_per, k_per)],
                    send_sem=send_sems.at[d],
                    recv_sem=recv_sems.at[d],
                    device_id=(j,),
                    device_id_type=pl.DeviceIdType.MESH,
                ).start()
            xg_ref[:, pl.ds(me * k_per, k_per)] = x_ref[
                pl.ds(me * m_per, m_per), :]
            for d in range(1, N_DEV):
                src = lax.rem(me - d + N_DEV, N_DEV)
                pltpu.make_async_remote_copy(
                    src_ref=x_ref.at[pl.ds(src * m_per, m_per), :],
                    dst_ref=xg_ref.at[:, pl.ds(src * k_per, k_per)],
                    send_sem=send_sems.at[d],
                    recv_sem=recv_sems.at[d],
                    device_id=(0,),
                    device_id_type=pl.DeviceIdType.MESH,
                ).wait_recv()

        @pl.when((t > 0) & (t + D - 1 < T))
        def _():
            c = t + D - 1
            w_dma(c, lax.rem(c, D)).start()

        slot = lax.rem(t, D)
        w_dma(t, slot).wait()
        o_ref[...] = jnp.dot(
            xg_ref[...], wring[slot].astype(jnp.bfloat16),
            preferred_element_type=jnp.float32,
        )

        @pl.when(t == T - 1)
        def _():
            for d in range(1, N_DEV):
                j = lax.rem(me + d, N_DEV)
                pltpu.make_async_remote_copy(
                    src_ref=x_ref.at[pl.ds(j * m_per, m_per), :],
                    dst_ref=xg_ref.at[:, pl.ds(me * k_per, k_per)],
                    send_sem=send_sems.at[d],
                    recv_sem=recv_sems.at[d],
                    device_id=(j,),
                    device_id_type=pl.DeviceIdType.MESH,
                ).wait_send()

    xb = x.astype(jnp.bfloat16)
    return pl.pallas_call(
        body,
        grid=(T,),
        out_shape=jax.ShapeDtypeStruct((m_per, n), jnp.float32),
        in_specs=[
            pl.BlockSpec((m, k_per), lambda t: (0, 0)),
            pl.BlockSpec(memory_space=pl.ANY),
        ],
        out_specs=pl.BlockSpec((m_per, nt), lambda t: (0, t)),
        scratch_shapes=[
            pltpu.VMEM((m_per, k), jnp.bfloat16),
            pltpu.VMEM((D, k, n // T), w_mat.dtype),
            pltpu.SemaphoreType.DMA((D,)),
            pltpu.SemaphoreType.DMA((N_DEV,)),
            pltpu.SemaphoreType.DMA((N_DEV,)),
        ],
        compiler_params=pltpu.CompilerParams(
            dimension_semantics=("arbitrary",),
            collective_id=0,
            vmem_limit_bytes=58 * 1024 * 1024,
        ),
    )(xb, w_mat)


# device time: 64895 ns/iter; 1.1036x vs baseline; 1.1036x over previous
import jax
import jax.numpy as jnp
from jax import lax
from jax.experimental import pallas as pl
from jax.experimental.pallas import tpu as pltpu

N_DEV = 16
T = 8
G = 2
D = 4

PERM = [0]
for _i in range(1, 8):
    PERM += [_i, N_DEV - _i]
PERM += [8]
assert sorted(PERM) == list(range(N_DEV))


def kernel(x, w_mat):
    m, k_per = x.shape
    k, n = w_mat.shape
    m_per = m // N_DEV
    nt = n // T
    kh = k // G
    n_chunks = G * T

    def body(x_ref, w_ref, o_ref, xb_ref, xg_ref, wring, dma_sems,
             send_sems, recv_sems):
        s = pl.program_id(0)
        me = lax.axis_index("i")

        def w_dmas(c, slot):
            g = c // T
            t = lax.rem(c, T)
            out = []
            for i in range(8):
                p = jnp.where(g == 0, PERM[i], PERM[8 + i])
                src = lax.rem(me + p, N_DEV)
                out.append(pltpu.make_async_copy(
                    w_ref.at[pl.ds(src * m_per, m_per),
                             pl.ds(t * nt, nt)],
                    wring.at[slot, pl.ds(i * m_per, m_per), :],
                    dma_sems.at[slot],
                ))
            return out

        def recv_rdma(i):
            src = lax.rem(me + PERM[i], N_DEV)
            return pltpu.make_async_remote_copy(
                src_ref=xb_ref.at[pl.ds(src * m_per, m_per), :],
                dst_ref=xg_ref.at[:, pl.ds(i * k_per, k_per)],
                send_sem=send_sems.at[i],
                recv_sem=recv_sems.at[i],
                device_id=(0,),
                device_id_type=pl.DeviceIdType.MESH,
            )

        def send_rdma(i):
            j = lax.rem(me - PERM[i] + N_DEV, N_DEV)
            return pltpu.make_async_remote_copy(
                src_ref=xb_ref.at[pl.ds(j * m_per, m_per), :],
                dst_ref=xg_ref.at[:, pl.ds(i * k_per, k_per)],
                send_sem=send_sems.at[i],
                recv_sem=recv_sems.at[i],
                device_id=(j,),
                device_id_type=pl.DeviceIdType.MESH,
            )

        @pl.when(s == 0)
        def _():
            xb_ref[...] = x_ref[...].astype(jnp.bfloat16)
            barrier_sem = pltpu.get_barrier_semaphore()
            for d in range(1, N_DEV):
                pl.semaphore_signal(
                    barrier_sem, inc=1,
                    device_id=(lax.rem(me + d, N_DEV),),
                    device_id_type=pl.DeviceIdType.MESH,
                )
            pl.semaphore_wait(barrier_sem, N_DEV - 1)
            for i in range(1, N_DEV):
                send_rdma(i).start()
            for c in range(D):
                for dma in w_dmas(c, c):
                    dma.start()
            xg_ref[:, pl.ds(0, k_per)] = xb_ref[pl.ds(me * m_per, m_per), :]
            for i in range(1, 8):
                recv_rdma(i).wait_recv()

        @pl.when(s == T)
        def _():
            for i in range(8, N_DEV):
                recv_rdma(i).wait_recv()

        @pl.when((s > 0) & (s + D - 1 < n_chunks))
        def _():
            c = s + D - 1
            for dma in w_dmas(c, lax.rem(c, D)):
                dma.start()

        slot = lax.rem(s, D)
        for dma in w_dmas(s, slot):
            dma.wait()

        g = s // T
        t = lax.rem(s, T)
        acc = jnp.dot(
            xg_ref[:, pl.ds(g * kh, kh)],
            wring[slot].astype(jnp.bfloat16),
            preferred_element_type=jnp.float32,
        )

        @pl.when(s < T)
        def _():
            o_ref[:, pl.ds(t * nt, nt)] = acc

        @pl.when(s >= T)
        def _():
            o_ref[:, pl.ds(t * nt, nt)] += acc

        @pl.when(s == n_chunks - 1)
        def _():
            for i in range(1, N_DEV):
                send_rdma(i).wait_send()

    return pl.pallas_call(
        body,
        grid=(n_chunks,),
        out_shape=jax.ShapeDtypeStruct((m_per, n), jnp.float32),
        in_specs=[
            pl.BlockSpec((m, k_per), lambda s: (0, 0)),
            pl.BlockSpec(memory_space=pl.ANY),
        ],
        out_specs=pl.BlockSpec((m_per, n), lambda s: (0, 0)),
        scratch_shapes=[
            pltpu.VMEM((m, k_per), jnp.bfloat16),
            pltpu.VMEM((m_per, k), jnp.bfloat16),
            pltpu.VMEM((D, kh, nt), w_mat.dtype),
            pltpu.SemaphoreType.DMA((D,)),
            pltpu.SemaphoreType.DMA((N_DEV,)),
            pltpu.SemaphoreType.DMA((N_DEV,)),
        ],
        compiler_params=pltpu.CompilerParams(
            dimension_semantics=("arbitrary",),
            collective_id=0,
            vmem_limit_bytes=60 * 1024 * 1024,
        ),
    )(x, w_mat)


# device time: 58460 ns/iter; 1.2251x vs baseline; 1.1101x over previous
import jax
import jax.numpy as jnp
from jax import lax
from jax.experimental import pallas as pl
from jax.experimental.pallas import tpu as pltpu

N_DEV = 16
T = 8
G = 2
D = 5

PERM = [0]
for _i in range(1, 8):
    PERM += [_i, N_DEV - _i]
PERM += [8]
assert sorted(PERM) == list(range(N_DEV))


def kernel(x, w_mat):
    m, k_per = x.shape
    k, n = w_mat.shape
    m_per = m // N_DEV
    nt = n // T
    kh = k // G
    n_chunks = G * T

    def body(x_ref, w_ref, o_ref, xb_ref, xg_ref, o_acc, wring, dma_sems,
             out_sems, send_sems, recv_sems):
        s = pl.program_id(0)
        me = lax.axis_index("i")

        def w_dmas(c, slot):
            g = c // T
            t = lax.rem(c, T)
            out = []
            for i in range(8):
                p = jnp.where(g == 0, PERM[i], PERM[8 + i])
                src = lax.rem(me + p, N_DEV)
                out.append(pltpu.make_async_copy(
                    w_ref.at[pl.ds(src * m_per, m_per),
                             pl.ds(t * nt, nt)],
                    wring.at[slot, pl.ds(i * m_per, m_per), :],
                    dma_sems.at[slot],
                ))
            return out

        def recv_rdma(i):
            src = lax.rem(me + PERM[i], N_DEV)
            return pltpu.make_async_remote_copy(
                src_ref=xb_ref.at[pl.ds(src * m_per, m_per), :],
                dst_ref=xg_ref.at[:, pl.ds(i * k_per, k_per)],
                send_sem=send_sems.at[i],
                recv_sem=recv_sems.at[i],
                device_id=(0,),
                device_id_type=pl.DeviceIdType.MESH,
            )

        def send_rdma(i):
            j = lax.rem(me - PERM[i] + N_DEV, N_DEV)
            return pltpu.make_async_remote_copy(
                src_ref=xb_ref.at[pl.ds(j * m_per, m_per), :],
                dst_ref=xg_ref.at[:, pl.ds(i * k_per, k_per)],
                send_sem=send_sems.at[i],
                recv_sem=recv_sems.at[i],
                device_id=(j,),
                device_id_type=pl.DeviceIdType.MESH,
            )

        @pl.when(s == 0)
        def _():
            for c in range(D):
                for dma in w_dmas(c, c):
                    dma.start()
            barrier_sem = pltpu.get_barrier_semaphore()
            for d in range(1, N_DEV):
                pl.semaphore_signal(
                    barrier_sem, inc=1,
                    device_id=(lax.rem(me + d, N_DEV),),
                    device_id_type=pl.DeviceIdType.MESH,
                )
            xb_ref[...] = x_ref[...].astype(jnp.bfloat16)
            pl.semaphore_wait(barrier_sem, N_DEV - 1)
            for i in range(1, N_DEV):
                send_rdma(i).start()
            xg_ref[:, pl.ds(0, k_per)] = xb_ref[pl.ds(me * m_per, m_per), :]
            for i in range(1, 8):
                recv_rdma(i).wait_recv()

        @pl.when(s == T)
        def _():
            for i in range(8, N_DEV):
                recv_rdma(i).wait_recv()

        @pl.when((s > 0) & (s + D - 1 < n_chunks))
        def _():
            c = s + D - 1
            for dma in w_dmas(c, lax.rem(c, D)):
                dma.start()

        slot = lax.rem(s, D)
        for dma in w_dmas(s, slot):
            dma.wait()

        g = s // T
        t = lax.rem(s, T)
        acc = jnp.dot(
            xg_ref[:, pl.ds(g * kh, kh)],
            wring[slot].astype(jnp.bfloat16),
            preferred_element_type=jnp.float32,
        )

        def o_dma(tt):
            return pltpu.make_async_copy(
                o_acc.at[:, pl.ds(tt * nt, nt)],
                o_ref.at[:, pl.ds(tt * nt, nt)],
                out_sems.at[tt],
            )

        @pl.when(s < T)
        def _():
            o_acc[:, pl.ds(t * nt, nt)] = acc

        @pl.when(s >= T)
        def _():
            o_acc[:, pl.ds(t * nt, nt)] += acc
            o_dma(t).start()

        @pl.when(s == n_chunks - 1)
        def _():
            for tt in range(T):
                o_dma(tt).wait()
            for i in range(1, N_DEV):
                send_rdma(i).wait_send()

    return pl.pallas_call(
        body,
        grid=(n_chunks,),
        out_shape=jax.ShapeDtypeStruct((m_per, n), jnp.float32),
        in_specs=[
            pl.BlockSpec((m, k_per), lambda s: (0, 0)),
            pl.BlockSpec(memory_space=pl.ANY),
        ],
        out_specs=pl.BlockSpec(memory_space=pl.ANY),
        scratch_shapes=[
            pltpu.VMEM((m, k_per), jnp.bfloat16),
            pltpu.VMEM((m_per, k), jnp.bfloat16),
            pltpu.VMEM((m_per, n), jnp.float32),
            pltpu.VMEM((D, kh, nt), w_mat.dtype),
            pltpu.SemaphoreType.DMA((D,)),
            pltpu.SemaphoreType.DMA((T,)),
            pltpu.SemaphoreType.DMA((N_DEV,)),
            pltpu.SemaphoreType.DMA((N_DEV,)),
        ],
        compiler_params=pltpu.CompilerParams(
            dimension_semantics=("arbitrary",),
            collective_id=0,
            vmem_limit_bytes=62 * 1024 * 1024,
        ),
    )(x, w_mat)
